# baseline (speedup 1.0000x reference)
import functools

import jax
import jax.numpy as jnp
from jax.experimental import pallas as pl
from jax.experimental.pallas import tpu as pltpu

_EPS = 1e-5


def _k1(x_ref, g_ref, b_ref, wa_ref, ba_ref, wb_ref, bb_ref, a_ref, bo_ref):
    x = x_ref[...]
    mu = jnp.mean(x, axis=1, keepdims=True)
    xc = x - mu
    var = jnp.mean(xc * xc, axis=1, keepdims=True)
    mh = xc * jax.lax.rsqrt(var + _EPS) * g_ref[...] + b_ref[...]
    a_ref[...] = (
        jnp.dot(mh, wa_ref[...], preferred_element_type=jnp.float32) + ba_ref[...]
    )
    bo_ref[...] = (
        jnp.dot(mh, wb_ref[...], preferred_element_type=jnp.float32) + bb_ref[...]
    )


def _k2(a_ref, b_ref, o_ref, *, scale):
    o_ref[...] = (
        jax.lax.dot_general(
            a_ref[...],
            b_ref[...],
            (((0,), (0,)), ((), ())),
            preferred_element_type=jnp.float32,
        )
        * scale
    )


def _k3(x_ref, w_ref, bias_ref, o_ref):
    o_ref[...] = (
        jnp.dot(x_ref[...], w_ref[...], preferred_element_type=jnp.float32)
        + bias_ref[...]
    )


@jax.jit
def kernel(m, ln_g, ln_b, Wa, ba, Wb, bb, Wo, bo):
    B, S, I, C = m.shape
    N = B * S * I
    BLK = 8192

    m2 = m.reshape(N, C)
    g2 = ln_g.reshape(1, C)
    b2 = ln_b.reshape(1, C)
    small = lambda shp: pl.BlockSpec(shp, lambda i: (0,) * len(shp))

    a2, bb2 = pl.pallas_call(
        _k1,
        grid=(N // BLK,),
        in_specs=[
            pl.BlockSpec((BLK, C), lambda i: (i, 0)),
            small((1, C)),
            small((1, C)),
            small((C, C)),
            small((1, C)),
            small((C, C)),
            small((1, C)),
        ],
        out_specs=[
            pl.BlockSpec((BLK, C), lambda i: (i, 0)),
            pl.BlockSpec((BLK, C), lambda i: (i, 0)),
        ],
        out_shape=[
            jax.ShapeDtypeStruct((N, C), jnp.float32),
            jax.ShapeDtypeStruct((N, C), jnp.float32),
        ],
        compiler_params=pltpu.CompilerParams(
            dimension_semantics=("parallel",),
        ),
    )(m2, g2, b2, Wa.T, ba.reshape(1, C), Wb.T, bb.reshape(1, C))

    a_t = a2.reshape(B, S, I, C).transpose(0, 3, 1, 2).reshape(B * C, S, I)
    b_t = bb2.reshape(B, S, I, C).transpose(0, 3, 1, 2).reshape(B * C, S, I)

    TJ = 256
    outer_t = pl.pallas_call(
        functools.partial(_k2, scale=1.0 / S),
        grid=(B * C, I // TJ),
        in_specs=[
            pl.BlockSpec((None, S, I), lambda bc, j: (bc, 0, 0)),
            pl.BlockSpec((None, S, TJ), lambda bc, j: (bc, 0, j)),
        ],
        out_specs=pl.BlockSpec((None, I, TJ), lambda bc, j: (bc, 0, j)),
        out_shape=jax.ShapeDtypeStruct((B * C, I, I), jnp.float32),
        compiler_params=pltpu.CompilerParams(
            dimension_semantics=("parallel", "parallel"),
        ),
    )(a_t, b_t)

    N2 = B * I * I
    outer_r = outer_t.reshape(B, C, I, I).transpose(0, 2, 3, 1).reshape(N2, C)

    out2 = pl.pallas_call(
        _k3,
        grid=(N2 // BLK,),
        in_specs=[
            pl.BlockSpec((BLK, C), lambda i: (i, 0)),
            small((C, C)),
            small((1, C)),
        ],
        out_specs=pl.BlockSpec((BLK, C), lambda i: (i, 0)),
        out_shape=jax.ShapeDtypeStruct((N2, C), jnp.float32),
        compiler_params=pltpu.CompilerParams(
            dimension_semantics=("parallel",),
        ),
    )(outer_r, Wo.T, bo.reshape(1, C))

    return out2.reshape(B, I, I, C)


# speedup vs baseline: 1.4006x; 1.4006x over previous
import jax
import jax.numpy as jnp
from jax.experimental import pallas as pl
from jax.experimental.pallas import tpu as pltpu

_EPS = 1e-5


def _k1(x_ref, g_ref, b_ref, w_ref, bias_ref, ab_ref):
    x = x_ref[...]
    mu = jnp.mean(x, axis=1, keepdims=True)
    xc = x - mu
    var = jnp.mean(xc * xc, axis=1, keepdims=True)
    mh = xc * jax.lax.rsqrt(var + _EPS) * g_ref[...] + b_ref[...]
    abt = jax.lax.dot_general(
        w_ref[...],
        mh,
        (((1,), (1,)), ((), ())),
        preferred_element_type=jnp.float32,
    )
    ab_ref[...] = (abt + bias_ref[...]).astype(jnp.bfloat16)


def _k2(a_ref, b_ref, o_ref):
    o_ref[...] = jax.lax.dot_general(
        a_ref[...],
        b_ref[...],
        (((0,), (0,)), ((), ())),
        preferred_element_type=jnp.float32,
    )


def _k3(x_ref, w_ref, bias_ref, o_ref):
    o_ref[...] = (
        jax.lax.dot_general(
            x_ref[...],
            w_ref[...],
            (((0,), (1,)), ((), ())),
            preferred_element_type=jnp.float32,
        )
        + bias_ref[...]
    )


@jax.jit
def kernel(m, ln_g, ln_b, Wa, ba, Wb, bb, Wo, bo):
    B, S, I, C = m.shape
    N = B * S * I
    BLK = 8192

    m2 = m.reshape(N, C)
    g2 = ln_g.reshape(1, C)
    b2 = ln_b.reshape(1, C)
    Wab = jnp.concatenate([Wa / S, Wb], axis=0)
    bias_ab = jnp.concatenate([ba / S, bb], axis=0).reshape(2 * C, 1)
    small = lambda shp: pl.BlockSpec(shp, lambda i: (0,) * len(shp))

    ab_t = pl.pallas_call(
        _k1,
        grid=(N // BLK,),
        in_specs=[
            pl.BlockSpec((BLK, C), lambda i: (i, 0)),
            small((1, C)),
            small((1, C)),
            small((2 * C, C)),
            small((2 * C, 1)),
        ],
        out_specs=pl.BlockSpec((2 * C, BLK), lambda i: (0, i)),
        out_shape=jax.ShapeDtypeStruct((2 * C, N), jnp.bfloat16),
        compiler_params=pltpu.CompilerParams(
            dimension_semantics=("parallel",),
        ),
    )(m2, g2, b2, Wab, bias_ab)

    ab5 = ab_t.reshape(2, C, B, S, I)

    TJ = 256
    outer_t = pl.pallas_call(
        _k2,
        grid=(C, B, I // TJ),
        in_specs=[
            pl.BlockSpec((None, None, None, S, I), lambda c, b, j: (0, c, b, 0, 0)),
            pl.BlockSpec((None, None, None, S, TJ), lambda c, b, j: (1, c, b, 0, j)),
        ],
        out_specs=pl.BlockSpec((None, None, I, TJ), lambda c, b, j: (c, b, 0, j)),
        out_shape=jax.ShapeDtypeStruct((C, B, I, I), jnp.float32),
        compiler_params=pltpu.CompilerParams(
            dimension_semantics=("parallel", "parallel", "parallel"),
        ),
    )(ab5, ab5)

    N2 = B * I * I
    outer2 = outer_t.reshape(C, N2)

    out2 = pl.pallas_call(
        _k3,
        grid=(N2 // BLK,),
        in_specs=[
            pl.BlockSpec((C, BLK), lambda i: (0, i)),
            small((C, C)),
            small((1, C)),
        ],
        out_specs=pl.BlockSpec((BLK, C), lambda i: (i, 0)),
        out_shape=jax.ShapeDtypeStruct((N2, C), jnp.float32),
        compiler_params=pltpu.CompilerParams(
            dimension_semantics=("parallel",),
        ),
    )(outer2, Wo, bo.reshape(1, C))

    return out2.reshape(B, I, I, C)


# speedup vs baseline: 2.4366x; 1.7398x over previous
import jax
import jax.numpy as jnp
from jax.experimental import pallas as pl
from jax.experimental.pallas import tpu as pltpu

_EPS = 1e-5
_RB = 16


def _k1(x_ref, g_ref, b_ref, w_ref, bias_ref, ab_ref):
    g = g_ref[...]
    b = b_ref[...]
    w = w_ref[...]
    bias = bias_ref[...]
    ncols = x_ref.shape[2]
    for r in range(_RB):
        x = x_ref[r]
        mu = jnp.mean(x, axis=0, keepdims=True)
        xc = x - mu
        var = jnp.mean(xc * xc, axis=0, keepdims=True)
        mh = xc * jax.lax.rsqrt(var + _EPS) * g + b
        y = jnp.dot(w, mh, preferred_element_type=jnp.float32) + bias
        ab_ref[:, r * ncols : (r + 1) * ncols] = y.astype(jnp.bfloat16)


def _k2(a_ref, b_ref, o_ref):
    o_ref[...] = jax.lax.dot_general(
        a_ref[...],
        b_ref[...],
        (((0,), (0,)), ((), ())),
        preferred_element_type=jnp.float32,
    )


def _k3(x_ref, w_ref, bias_ref, o_ref):
    w = w_ref[...]
    bias = bias_ref[...]
    ncols = o_ref.shape[2]
    for r in range(_RB):
        x = x_ref[:, r * ncols : (r + 1) * ncols]
        o_ref[r] = jnp.dot(w, x, preferred_element_type=jnp.float32) + bias


@jax.jit
def kernel(m, ln_g, ln_b, Wa, ba, Wb, bb, Wo, bo):
    B, S, I, C = m.shape
    BS = B * S

    mt = jnp.transpose(m, (0, 1, 3, 2)).reshape(BS, C, I)
    g2 = ln_g.reshape(C, 1)
    b2 = ln_b.reshape(C, 1)
    Wab = jnp.concatenate([Wa / S, Wb], axis=0)
    bias_ab = jnp.concatenate([ba / S, bb], axis=0).reshape(2 * C, 1)
    small = lambda shp: pl.BlockSpec(shp, lambda *_: (0,) * len(shp))

    ab = pl.pallas_call(
        _k1,
        grid=(BS // _RB,),
        in_specs=[
            pl.BlockSpec((_RB, C, I), lambda i: (i, 0, 0)),
            small((C, 1)),
            small((C, 1)),
            small((2 * C, C)),
            small((2 * C, 1)),
        ],
        out_specs=pl.BlockSpec((2 * C, _RB * I), lambda i: (0, i)),
        out_shape=jax.ShapeDtypeStruct((2 * C, BS * I), jnp.bfloat16),
        compiler_params=pltpu.CompilerParams(
            dimension_semantics=("parallel",),
        ),
    )(mt, g2, b2, Wab, bias_ab)

    ab4 = ab.reshape(2 * C, B, S, I)

    TJ = 256
    outer_t = pl.pallas_call(
        _k2,
        grid=(C, B, I // TJ),
        in_specs=[
            pl.BlockSpec((None, None, S, I), lambda c, b, j: (c, b, 0, 0)),
            pl.BlockSpec((None, None, S, TJ), lambda c, b, j: (C + c, b, 0, j)),
        ],
        out_specs=pl.BlockSpec((None, None, I, TJ), lambda c, b, j: (c, b, 0, j)),
        out_shape=jax.ShapeDtypeStruct((C, B, I, I), jnp.float32),
        compiler_params=pltpu.CompilerParams(
            dimension_semantics=("parallel", "parallel", "parallel"),
        ),
    )(ab4, ab4)

    BI = B * I
    outer2 = outer_t.reshape(C, BI * I)

    res = pl.pallas_call(
        _k3,
        grid=(BI // _RB,),
        in_specs=[
            pl.BlockSpec((C, _RB * I), lambda i: (0, i)),
            small((C, C)),
            small((C, 1)),
        ],
        out_specs=pl.BlockSpec((_RB, C, I), lambda i: (i, 0, 0)),
        out_shape=jax.ShapeDtypeStruct((BI, C, I), jnp.float32),
        compiler_params=pltpu.CompilerParams(
            dimension_semantics=("parallel",),
        ),
    )(outer2, Wo, bo.reshape(C, 1))

    return jnp.transpose(res.reshape(B, I, C, I), (0, 1, 3, 2))


# speedup vs baseline: 3.1305x; 1.2847x over previous
import jax
import jax.numpy as jnp
from jax.experimental import pallas as pl
from jax.experimental.pallas import tpu as pltpu

_EPS = 1e-5
_RB = 16


def _k1(x_ref, g_ref, b_ref, l_ref, bias_ref, ab_ref):
    x3 = x_ref[...]
    rb, c, ncols = x3.shape
    mu = jnp.mean(x3, axis=1, keepdims=True)
    xc = x3 - mu
    var = jnp.mean(xc * xc, axis=1, keepdims=True)
    g3 = g_ref[...][None]
    b3 = b_ref[...][None]
    mh = xc * jax.lax.rsqrt(var + _EPS) * g3 + b3
    x2 = mh.reshape(rb * c, ncols)
    z = jnp.dot(l_ref[...], x2, preferred_element_type=jnp.float32) + bias_ref[...]
    ab_ref[...] = z.astype(jnp.bfloat16).reshape(2 * c, rb, ncols)


def _k2(a_ref, b_ref, o_ref):
    o_ref[...] = jax.lax.dot_general(
        a_ref[...],
        b_ref[...],
        (((0,), (0,)), ((), ())),
        preferred_element_type=jnp.float32,
    )


def _k3(x_ref, l_ref, bias_ref, o_ref):
    x3 = x_ref[...]
    c, rb, ncols = x3.shape
    x2 = x3.reshape(c * rb, ncols)
    z = jnp.dot(l_ref[...], x2, preferred_element_type=jnp.float32) + bias_ref[...]
    o_ref[...] = z.reshape(rb, c, ncols)


@jax.jit
def kernel(m, ln_g, ln_b, Wa, ba, Wb, bb, Wo, bo):
    B, S, I, C = m.shape

    mt = jnp.transpose(m, (0, 1, 3, 2))
    g2 = ln_g.reshape(C, 1)
    b2 = ln_b.reshape(C, 1)
    Wab = jnp.concatenate([Wa / S, Wb], axis=0)
    L1 = (Wab[:, None, None, :] * jnp.eye(_RB)[None, :, :, None]).reshape(
        2 * C * _RB, _RB * C
    )
    bias1 = jnp.repeat(jnp.concatenate([ba / S, bb]), _RB).reshape(2 * C * _RB, 1)
    L3 = (jnp.eye(_RB)[:, None, None, :] * Wo[None, :, :, None]).reshape(
        _RB * C, C * _RB
    )
    bias3 = jnp.tile(bo, _RB).reshape(_RB * C, 1)
    small = lambda shp: pl.BlockSpec(shp, lambda *_: (0,) * len(shp))

    ab = pl.pallas_call(
        _k1,
        grid=(B, S // _RB),
        in_specs=[
            pl.BlockSpec((None, _RB, C, I), lambda b, s: (b, s, 0, 0)),
            small((C, 1)),
            small((C, 1)),
            small((2 * C * _RB, _RB * C)),
            small((2 * C * _RB, 1)),
        ],
        out_specs=pl.BlockSpec((2 * C, None, _RB, I), lambda b, s: (0, b, s, 0)),
        out_shape=jax.ShapeDtypeStruct((2 * C, B, S, I), jnp.bfloat16),
        compiler_params=pltpu.CompilerParams(
            dimension_semantics=("parallel", "parallel"),
        ),
    )(mt, g2, b2, L1, bias1)

    TJ = 256
    outer_t = pl.pallas_call(
        _k2,
        grid=(C, B, I // TJ),
        in_specs=[
            pl.BlockSpec((None, None, S, I), lambda c, b, j: (c, b, 0, 0)),
            pl.BlockSpec((None, None, S, TJ), lambda c, b, j: (C + c, b, 0, j)),
        ],
        out_specs=pl.BlockSpec((None, None, I, TJ), lambda c, b, j: (c, b, 0, j)),
        out_shape=jax.ShapeDtypeStruct((C, B, I, I), jnp.float32),
        compiler_params=pltpu.CompilerParams(
            dimension_semantics=("parallel", "parallel", "parallel"),
        ),
    )(ab, ab)

    res = pl.pallas_call(
        _k3,
        grid=(B, I // _RB),
        in_specs=[
            pl.BlockSpec((C, None, _RB, I), lambda b, i: (0, b, i, 0)),
            small((_RB * C, C * _RB)),
            small((_RB * C, 1)),
        ],
        out_specs=pl.BlockSpec((None, _RB, C, I), lambda b, i: (b, i, 0, 0)),
        out_shape=jax.ShapeDtypeStruct((B, I, C, I), jnp.float32),
        compiler_params=pltpu.CompilerParams(
            dimension_semantics=("parallel", "parallel"),
        ),
    )(outer_t, L3, bias3)

    return jnp.transpose(res, (0, 1, 3, 2))
